# baseline (speedup 1.0000x reference)
import functools

import jax
import jax.numpy as jnp
from jax import lax
from jax.experimental import pallas as pl
from jax.experimental.pallas import tpu as pltpu
from jax.experimental.pallas import tpu_sc as plsc

N = 200
E = 12800
NFEAT = 512
NHID = 256
NCLASS = 128

HALF = 25600
SC_SZ = 2 * HALF
EPT = E // 16
IDX_ROWS = 13
NVAL = IDX_ROWS * 128
CNT_COL = 72
DUMP = HALF + 127
SLICE = SC_SZ // 16

_sc_mesh = plsc.VectorSubcoreMesh(core_axis_name="c", subcore_axis_name="s")


@functools.partial(
    pl.kernel,
    mesh=_sc_mesh,
    out_type=jax.ShapeDtypeStruct((2 * SC_SZ,), jnp.float32),
    scratch_types=[
        pltpu.VMEM((EPT,), jnp.int32),
        pltpu.VMEM((EPT,), jnp.int32),
        pltpu.VMEM((NVAL,), jnp.float32),
        [pltpu.VMEM((128,), jnp.int32) for _ in range(IDX_ROWS)],
        pltpu.VMEM((SLICE,), jnp.float32),
        pltpu.VMEM_SHARED((SC_SZ,), jnp.float32),
        pltpu.SemaphoreType.DMA,
        pltpu.SemaphoreType.DMA,
    ],
)
def _adj_sc(sdS, eaS, out,
            src_v, dst_v, vals_v, idx_refs, zbuf, acc_sh, sem, sem2):
    cid = lax.axis_index("c")
    sid = lax.axis_index("s")
    ebase = cid * E + sid * EPT
    row0 = sid * SLICE

    cps = [
        pltpu.async_copy(sdS.at[pl.ds(ebase, EPT)], src_v, sem),
        pltpu.async_copy(sdS.at[pl.ds(2 * E + ebase, EPT)], dst_v, sem),
        pltpu.async_copy(eaS.at[pl.ds(ebase, EPT)],
                         vals_v.at[pl.ds(0, EPT)], sem),
    ]
    zero16 = jnp.zeros((16,), jnp.float32)
    one16 = jnp.ones((16,), jnp.float32)
    for i in range(SLICE // 16):
        zbuf[pl.ds(i * 16, 16)] = zero16
    for i in range(EPT, NVAL, 16):
        vals_v[pl.ds(i, 16)] = one16
    for c in cps:
        c.wait()
    pltpu.sync_copy(zbuf, acc_sh.at[pl.ds(row0, SLICE)])
    for g in range(EPT // 16):
        s = src_v[pl.ds(g * 16, 16)]
        d = dst_v[pl.ds(g * 16, 16)]
        p = g * 16
        hi = jnp.where(s >= 128, jnp.int32(HALF - 128), jnp.int32(0))
        idx_refs[p // 128][pl.ds(p % 128, 16)] = hi + d * 128 + s
    for g in range(EPT // 16):
        d = dst_v[pl.ds(g * 16, 16)]
        p = EPT + g * 16
        idx_refs[p // 128][pl.ds(p % 128, 16)] = d * 128 + (HALF + CNT_COL)
    for p in range(2 * EPT, NVAL, 16):
        idx_refs[p // 128][pl.ds(p % 128, 16)] = jnp.full((16,), DUMP,
                                                          jnp.int32)
    plsc.subcore_barrier()
    scs = [
        pltpu.async_copy(vals_v.at[pl.ds(j * 128, 128)],
                         acc_sh.at[idx_refs[j]], sem2, add=True)
        for j in range(IDX_ROWS)
    ]
    for c in scs:
        c.wait()
    plsc.subcore_barrier()
    pltpu.sync_copy(acc_sh.at[pl.ds(row0, SLICE)],
                    out.at[pl.ds(cid * SC_SZ + row0, SLICE)])


def _tdot(a, b):
    return lax.dot_general(a, b, (((0,), (0,)), ((), ())),
                           preferred_element_type=jnp.float32)


def _mm(a, b):
    return jnp.dot(a, b, preferred_element_type=jnp.float32)


def _pre_kernel(x1_ref, x2_ref, Ws1_ref, b1_ref, xs1_ref, xs2_ref):
    xs1_ref[...] = _mm(x1_ref[...], Ws1_ref[...]) + b1_ref[...]
    xs2_ref[...] = _mm(x2_ref[...], Ws1_ref[...]) + b1_ref[...]


def _branch(xa_ref, xb_ref, buf_ref, b0, xs_ref, Wn1_ref, Wn2_ref,
            Ws2_ref, b2_ref):
    alo = buf_ref[b0:b0 + N, :]
    ahi = buf_ref[b0 + N:b0 + 2 * N, :]
    sel = (lax.broadcasted_iota(jnp.int32, (128, 1), 0)
           == CNT_COL).astype(jnp.float32)
    cnt = _mm(ahi, sel)
    inv = 1.0 / jnp.maximum(cnt, 1.0)
    xa = xa_ref[...]
    xb = xb_ref[...]
    agg1 = (_mm(alo, xa) + _mm(ahi, xb)) * inv
    h = jax.nn.relu(_mm(agg1, Wn1_ref[...]) + xs_ref[...])
    ha = h[:128, :]
    hb = jnp.concatenate([h[128:, :], jnp.zeros((256 - N, NHID),
                                                jnp.float32)], axis=0)
    agg2 = (_mm(alo, ha) + _mm(ahi, hb)) * inv
    return _mm(agg2, Wn2_ref[...]) + _mm(h, Ws2_ref[...]) + b2_ref[...]


def _classifier(o, Wc1_ref, bc1_ref, Wc2_ref, bc2_ref, Wc3_ref, bc3_ref):
    t = jax.nn.relu(_tdot(o, Wc1_ref[...]) + bc1_ref[...])
    t = jax.nn.relu(_mm(t, Wc2_ref[...]) + bc2_ref[...])
    return _mm(t, Wc3_ref[...]) + bc3_ref[...]


def _dense_kernel(x1a_ref, xb1_ref, x2a_ref, xb2_ref, buf_ref, xs1_ref,
                  xs2_ref, Wn1_ref, Wn2_ref, Ws2_ref, b2_ref,
                  Wc1_ref, bc1_ref, Wc2_ref, bc2_ref, Wc3_ref, bc3_ref,
                  out1_ref, out2_ref):
    o1 = _branch(x1a_ref, xb1_ref, buf_ref, 0, xs1_ref, Wn1_ref,
                 Wn2_ref, Ws2_ref, b2_ref)
    o2 = _branch(x2a_ref, xb2_ref, buf_ref, 2 * N, xs2_ref, Wn1_ref,
                 Wn2_ref, Ws2_ref, b2_ref)
    out1_ref[...] = _classifier(o1, Wc1_ref, bc1_ref, Wc2_ref, bc2_ref,
                                Wc3_ref, bc3_ref)
    out2_ref[...] = _classifier(o2, Wc1_ref, bc1_ref, Wc2_ref, bc2_ref,
                                Wc3_ref, bc3_ref)


@jax.jit
def kernel(x1, edge_index1, edge_attr1, x2, edge_index2, edge_attr2,
           Wn1, Ws1, b1, Wn2, Ws2, b2, Wc1, bc1, Wc2, bc2, Wc3, bc3):
    sdS = jnp.concatenate([edge_index1[0], edge_index2[0],
                           edge_index1[1], edge_index2[1]])
    eaS = jnp.concatenate([edge_attr1, edge_attr2])
    out_flat = _adj_sc(sdS, eaS)
    buf = out_flat.reshape(4 * N, 128)

    xb1 = jnp.pad(x1[128:, :], ((0, 256 - N), (0, 0)))
    xb2 = jnp.pad(x2[128:, :], ((0, 256 - N), (0, 0)))
    x1a = x1[:128, :]
    x2a = x2[:128, :]

    xs1, xs2 = pl.pallas_call(
        _pre_kernel,
        out_shape=(
            jax.ShapeDtypeStruct((N, NHID), jnp.float32),
            jax.ShapeDtypeStruct((N, NHID), jnp.float32),
        ),
    )(x1, x2, Ws1, b1.reshape(1, NHID))

    out1, out2 = pl.pallas_call(
        _dense_kernel,
        out_shape=(
            jax.ShapeDtypeStruct((NCLASS, 10), jnp.float32),
            jax.ShapeDtypeStruct((NCLASS, 10), jnp.float32),
        ),
    )(x1a, xb1, x2a, xb2, buf, xs1, xs2,
      Wn1, Wn2, Ws2, b2.reshape(1, NCLASS),
      Wc1, bc1.reshape(1, 100), Wc2, bc2.reshape(1, 50), Wc3,
      bc3.reshape(1, 10))
    return out1, out2
